# baseline (speedup 1.0000x reference)
import functools

import jax
import jax.numpy as jnp
from jax import lax
from jax.experimental import pallas as pl
from jax.experimental.pallas import tpu as pltpu
from jax.experimental.pallas import tpu_sc as plsc

N_NODES = 10000
N_EDGES = 320000
D = 128
NUM_LAYERS = 3

NC = 2
NS = 16
CHUNK = 128
CPW = 80
EPW = CPW * CHUNK
E_PAD = NC * NS * EPW
N_PAD = 10240
ZROWS = N_PAD // NS
ZCH = ZROWS // CHUNK


def _sc_agg_body(
    src_hbm, dst_hbm, h_hbm, out_hbm,
    dstv, sidx, buf0, buf1, agg_sh,
    g0, g1, i0, i1, i2, i3,
):
    isems = (i0, i1, i2, i3)
    gbufs = (buf0, buf1)
    gsems = (g0, g1)
    c = lax.axis_index("c")
    s = lax.axis_index("s")
    w = c * NS + s

    def _idx_start(j, r):
        pltpu.async_copy(src_hbm.at[w].at[j], sidx.at[r], isems[r])

    def _idx_wait(j, r):
        pltpu.make_async_copy(src_hbm.at[w].at[j], sidx.at[r], isems[r]).wait()

    def _g_start(r, b):
        pltpu.async_copy(h_hbm.at[sidx.at[r]], gbufs[b], gsems[b])

    def _g_wait(r, b):
        pltpu.make_async_copy(h_hbm.at[sidx.at[r]], gbufs[b], gsems[b]).wait()

    pltpu.sync_copy(dst_hbm.at[w], dstv)
    for j in range(4):
        _idx_start(j, j)

    def _zero(k, _):
        i = k // (D // 16)
        j = k % (D // 16)
        buf0[i, pl.ds(j * 16, 16)] = jnp.zeros((16,), jnp.float32)
        return 0

    lax.fori_loop(0, CHUNK * (D // 16), _zero, 0)
    for z in range(ZCH):
        pltpu.sync_copy(buf0, agg_sh.at[pl.ds(s * ZROWS + z * CHUNK, CHUNK)])
    plsc.subcore_barrier()

    for j in range(2):
        _idx_wait(j, j)
        _g_start(j, j)

    def _step(j, p, do_idx, do_g):
        b = p % 2
        r = p % 4
        _g_wait(r, b)
        pltpu.sync_copy(gbufs[b], agg_sh.at[dstv.at[j]], add=True)
        if do_idx:
            _idx_start(j + 4, r)
        if do_g:
            r2 = (p + 2) % 4
            _idx_wait(j + 2, r2)
            _g_start(r2, b)

    def _quad(g, _):
        j0 = 4 * g
        for p in range(4):
            _step(j0 + p, p, True, True)
        return 0

    lax.fori_loop(0, (CPW - 4) // 4, _quad, 0)
    _step(CPW - 4, 0, False, True)
    _step(CPW - 3, 1, False, True)
    _step(CPW - 2, 2, False, False)
    _step(CPW - 1, 3, False, False)
    plsc.subcore_barrier()

    for z in range(ZCH):
        r0 = s * ZROWS + z * CHUNK
        pltpu.sync_copy(agg_sh.at[pl.ds(r0, CHUNK)], buf0)
        pltpu.sync_copy(buf0, out_hbm.at[c].at[pl.ds(r0, CHUNK)])


_sc_agg = pl.kernel(
    _sc_agg_body,
    out_type=jax.ShapeDtypeStruct((NC, N_PAD, D), jnp.float32),
    mesh=plsc.VectorSubcoreMesh(
        core_axis_name="c", subcore_axis_name="s", num_cores=NC, num_subcores=NS
    ),
    scratch_types=[
        pltpu.VMEM((CPW, CHUNK), jnp.int32),
        pltpu.VMEM((4, CHUNK), jnp.int32),
        pltpu.VMEM((CHUNK, D), jnp.float32),
        pltpu.VMEM((CHUNK, D), jnp.float32),
        pltpu.VMEM_SHARED((N_PAD, D), jnp.float32),
        pltpu.SemaphoreType.DMA,
        pltpu.SemaphoreType.DMA,
        pltpu.SemaphoreType.DMA,
        pltpu.SemaphoreType.DMA,
        pltpu.SemaphoreType.DMA,
        pltpu.SemaphoreType.DMA,
    ],
)


def _mlp_body(eps_ref, h_ref, a0_ref, a1_ref, w1_ref, b1_ref, w2_ref, b2_ref, o_ref):
    rst = h_ref[...] * (1.0 + eps_ref[0, 0]) + a0_ref[...] + a1_ref[...]
    hid = jnp.maximum(
        jnp.dot(rst, w1_ref[...], preferred_element_type=jnp.float32) + b1_ref[...], 0.0
    )
    o_ref[...] = jnp.dot(hid, w2_ref[...], preferred_element_type=jnp.float32) + b2_ref[...]


def _mlp_sum_body(eps_ref, h_ref, a0_ref, a1_ref, w1_ref, b1_ref, w2_ref, b2_ref, o_ref):
    rst = h_ref[...] * (1.0 + eps_ref[0, 0]) + a0_ref[...] + a1_ref[...]
    hid = jnp.maximum(
        jnp.dot(rst, w1_ref[...], preferred_element_type=jnp.float32) + b1_ref[...], 0.0
    )
    out = jnp.dot(hid, w2_ref[...], preferred_element_type=jnp.float32) + b2_ref[...]

    @pl.when(pl.program_id(0) == 0)
    def _():
        o_ref[...] = jnp.zeros_like(o_ref)

    o_ref[...] += jnp.sum(out, axis=0, keepdims=True)


_MLP_BLOCK = 1000
_MLP_GRID = N_NODES // _MLP_BLOCK


def _mlp_call(body, out_shape, out_spec):
    return pl.pallas_call(
        body,
        grid=(_MLP_GRID,),
        in_specs=[
            pl.BlockSpec(memory_space=pltpu.SMEM),
            pl.BlockSpec((_MLP_BLOCK, D), lambda i: (i, 0)),
            pl.BlockSpec((_MLP_BLOCK, D), lambda i: (i, 0)),
            pl.BlockSpec((_MLP_BLOCK, D), lambda i: (i, 0)),
            pl.BlockSpec((D, D), lambda i: (0, 0)),
            pl.BlockSpec((1, D), lambda i: (0, 0)),
            pl.BlockSpec((D, D), lambda i: (0, 0)),
            pl.BlockSpec((1, D), lambda i: (0, 0)),
        ],
        out_specs=out_spec,
        out_shape=out_shape,
    )


_mlp = _mlp_call(
    _mlp_body,
    jax.ShapeDtypeStruct((N_NODES, D), jnp.float32),
    pl.BlockSpec((_MLP_BLOCK, D), lambda i: (i, 0)),
)
_mlp_sum = _mlp_call(
    _mlp_sum_body,
    jax.ShapeDtypeStruct((1, D), jnp.float32),
    pl.BlockSpec((1, D), lambda i: (0, 0)),
)


@jax.jit
def kernel(feats, edge_index, W1, b1, W2, b2, eps):
    src = edge_index[0].astype(jnp.int32)
    dst = edge_index[1].astype(jnp.int32)
    pad = E_PAD - N_EDGES
    src = jnp.concatenate([src, jnp.zeros((pad,), jnp.int32)]).reshape(NC * NS, CPW, CHUNK)
    dst = jnp.concatenate([dst, jnp.full((pad,), N_NODES, jnp.int32)]).reshape(
        NC * NS, CPW, CHUNK
    )

    h = feats
    for i in range(NUM_LAYERS):
        agg = _sc_agg(src, dst, h)
        a0 = agg[0, :N_NODES]
        a1 = agg[1, :N_NODES]
        eps_i = eps[i].reshape(1, 1)
        args = (eps_i, h, a0, a1, W1[i], b1[i].reshape(1, D), W2[i], b2[i].reshape(1, D))
        if i < NUM_LAYERS - 1:
            h = _mlp(*args)
        else:
            return _mlp_sum(*args)


# speedup vs baseline: 3.8898x; 3.8898x over previous
import functools

import jax
import jax.numpy as jnp
from jax import lax
from jax.experimental import pallas as pl
from jax.experimental.pallas import tpu as pltpu
from jax.experimental.pallas import tpu_sc as plsc

N_NODES = 10000
N_EDGES = 320000
D = 128
NUM_LAYERS = 3

NC = 2
NS = 16
CHUNK = 128
CPW = 80
EPW = CPW * CHUNK
E_PAD = NC * NS * EPW
N_PAD = 10240
ZROWS = N_PAD // NS
ZCH = ZROWS // CHUNK


def _sc_agg_body(
    src_hbm, dst_hbm, h_hbm, out_hbm,
    dstv, sidx, buf0, buf1, agg_sh,
    g0, g1, i0, i1, i2, i3,
):
    isems = (i0, i1, i2, i3)
    gbufs = (buf0, buf1)
    gsems = (g0, g1)
    c = lax.axis_index("c")
    s = lax.axis_index("s")
    w = c * NS + s

    def _idx_start(j, r):
        pltpu.async_copy(src_hbm.at[w].at[j], sidx.at[r], isems[r])

    def _idx_wait(j, r):
        pltpu.make_async_copy(src_hbm.at[w].at[j], sidx.at[r], isems[r]).wait()

    def _g_start(r, b):
        pltpu.async_copy(h_hbm.at[sidx.at[r]], gbufs[b], gsems[b])

    def _g_wait(r, b):
        pltpu.make_async_copy(h_hbm.at[sidx.at[r]], gbufs[b], gsems[b]).wait()

    pltpu.sync_copy(dst_hbm.at[w], dstv)
    for j in range(4):
        _idx_start(j, j)

    def _zero(k, _):
        i = k // (D // 16)
        j = k % (D // 16)
        buf0[i, pl.ds(j * 16, 16)] = jnp.zeros((16,), jnp.float32)
        return 0

    lax.fori_loop(0, CHUNK * (D // 16), _zero, 0)
    for z in range(ZCH):
        pltpu.sync_copy(buf0, agg_sh.at[pl.ds(s * ZROWS + z * CHUNK, CHUNK)])
    plsc.subcore_barrier()

    for j in range(2):
        _idx_wait(j, j)
        _g_start(j, j)

    def _step(j, p, do_idx, do_g):
        b = p % 2
        r = p % 4
        _g_wait(r, b)
        pltpu.sync_copy(gbufs[b], agg_sh.at[dstv.at[j]], add=True)
        if do_idx:
            _idx_start(j + 4, r)
        if do_g:
            r2 = (p + 2) % 4
            _idx_wait(j + 2, r2)
            _g_start(r2, b)

    def _quad(g, _):
        j0 = 4 * g
        for p in range(4):
            _step(j0 + p, p, True, True)
        return 0

    lax.fori_loop(0, (CPW - 4) // 4, _quad, 0)
    _step(CPW - 4, 0, False, True)
    _step(CPW - 3, 1, False, True)
    _step(CPW - 2, 2, False, False)
    _step(CPW - 1, 3, False, False)
    plsc.subcore_barrier()

    for z in range(ZCH):
        r0 = s * ZROWS + z * CHUNK
        pltpu.sync_copy(agg_sh.at[pl.ds(r0, CHUNK)], buf0)
        pltpu.sync_copy(buf0, out_hbm.at[c].at[pl.ds(r0, CHUNK)])


_sc_agg = pl.kernel(
    _sc_agg_body,
    out_type=jax.ShapeDtypeStruct((NC, N_PAD, D), jnp.float32),
    mesh=plsc.VectorSubcoreMesh(
        core_axis_name="c", subcore_axis_name="s", num_cores=NC, num_subcores=NS
    ),
    scratch_types=[
        pltpu.VMEM((CPW, CHUNK), jnp.int32),
        pltpu.VMEM((4, CHUNK), jnp.int32),
        pltpu.VMEM((CHUNK, D), jnp.float32),
        pltpu.VMEM((CHUNK, D), jnp.float32),
        pltpu.VMEM_SHARED((N_PAD, D), jnp.float32),
        pltpu.SemaphoreType.DMA,
        pltpu.SemaphoreType.DMA,
        pltpu.SemaphoreType.DMA,
        pltpu.SemaphoreType.DMA,
        pltpu.SemaphoreType.DMA,
        pltpu.SemaphoreType.DMA,
    ],
)


def _mlp_body(eps_ref, h_ref, a0_ref, a1_ref, w1_ref, b1_ref, w2_ref, b2_ref, o_ref):
    rst = h_ref[...] * (1.0 + eps_ref[0, 0]) + a0_ref[0] + a1_ref[0]
    hid = jnp.maximum(
        jnp.dot(rst, w1_ref[...], preferred_element_type=jnp.float32) + b1_ref[...], 0.0
    )
    o_ref[...] = jnp.dot(hid, w2_ref[...], preferred_element_type=jnp.float32) + b2_ref[...]


def _mlp_sum_body(eps_ref, h_ref, a0_ref, a1_ref, w1_ref, b1_ref, w2_ref, b2_ref, o_ref):
    rst = h_ref[...] * (1.0 + eps_ref[0, 0]) + a0_ref[0] + a1_ref[0]
    hid = jnp.maximum(
        jnp.dot(rst, w1_ref[...], preferred_element_type=jnp.float32) + b1_ref[...], 0.0
    )
    out = jnp.dot(hid, w2_ref[...], preferred_element_type=jnp.float32) + b2_ref[...]

    @pl.when(pl.program_id(0) == 0)
    def _():
        o_ref[...] = jnp.zeros_like(o_ref)

    o_ref[...] += jnp.sum(out, axis=0, keepdims=True)


_MLP_BLOCK = 1000
_MLP_GRID = N_NODES // _MLP_BLOCK


def _mlp_call(body, out_shape, out_spec):
    return pl.pallas_call(
        body,
        grid=(_MLP_GRID,),
        in_specs=[
            pl.BlockSpec(memory_space=pltpu.SMEM),
            pl.BlockSpec((_MLP_BLOCK, D), lambda i: (i, 0)),
            pl.BlockSpec((1, _MLP_BLOCK, D), lambda i: (0, i, 0)),
            pl.BlockSpec((1, _MLP_BLOCK, D), lambda i: (1, i, 0)),
            pl.BlockSpec((D, D), lambda i: (0, 0)),
            pl.BlockSpec((1, D), lambda i: (0, 0)),
            pl.BlockSpec((D, D), lambda i: (0, 0)),
            pl.BlockSpec((1, D), lambda i: (0, 0)),
        ],
        out_specs=out_spec,
        out_shape=out_shape,
    )


_mlp = _mlp_call(
    _mlp_body,
    jax.ShapeDtypeStruct((N_NODES, D), jnp.float32),
    pl.BlockSpec((_MLP_BLOCK, D), lambda i: (i, 0)),
)
_mlp_sum = _mlp_call(
    _mlp_sum_body,
    jax.ShapeDtypeStruct((1, D), jnp.float32),
    pl.BlockSpec((1, D), lambda i: (0, 0)),
)


@jax.jit
def kernel(feats, edge_index, W1, b1, W2, b2, eps):
    src = edge_index[0].astype(jnp.int32)
    dst = edge_index[1].astype(jnp.int32)
    epc = NS * EPW
    rpc = N_EDGES // NC
    ppc = epc - rpc
    pad_src = jnp.arange(ppc, dtype=jnp.int32) % N_NODES
    pad_dst = jnp.arange(ppc, dtype=jnp.int32) % (N_PAD - N_NODES) + N_NODES
    src = jnp.concatenate(
        [src[:rpc], pad_src, src[rpc:], pad_src]
    ).reshape(NC * NS, CPW, CHUNK)
    dst = jnp.concatenate(
        [dst[:rpc], pad_dst, dst[rpc:], pad_dst]
    ).reshape(NC * NS, CPW, CHUNK)

    h = feats
    for i in range(NUM_LAYERS):
        agg = _sc_agg(src, dst, h)
        eps_i = eps[i].reshape(1, 1)
        args = (eps_i, h, agg, agg, W1[i], b1[i].reshape(1, D), W2[i], b2[i].reshape(1, D))
        if i < NUM_LAYERS - 1:
            h = _mlp(*args)
        else:
            return _mlp_sum(*args)


# speedup vs baseline: 4.3308x; 1.1134x over previous
import functools

import jax
import jax.numpy as jnp
from jax import lax
from jax.experimental import pallas as pl
from jax.experimental.pallas import tpu as pltpu
from jax.experimental.pallas import tpu_sc as plsc

N_NODES = 10000
N_EDGES = 320000
D = 128
NUM_LAYERS = 3

NC = 2
NS = 16
CHUNK = 128
CPW = 80
EPW = CPW * CHUNK
E_PAD = NC * NS * EPW
N_PAD = 10112
ZROWS = N_PAD // NS
ZSPANS = ((0, 128), (128, 128), (256, 128), (384, 128), (512, 120))


def _sc_agg_body(
    src_hbm, dst_hbm, h_hbm, out_hbm,
    didx, sidx, buf0, buf1, buf2, agg_sh,
    g0, g1, g2, i0, i1, i2, i3, d0, d1, d2,
):
    isems = (i0, i1, i2, i3)
    dsems = (d0, d1, d2)
    gbufs = (buf0, buf1, buf2)
    gsems = (g0, g1, g2)
    c = lax.axis_index("c")
    s = lax.axis_index("s")
    w = c * NS + s

    def _sidx_start(j, r):
        pltpu.async_copy(src_hbm.at[w].at[j], sidx.at[r], isems[r])

    def _sidx_wait(j, r):
        pltpu.make_async_copy(src_hbm.at[w].at[j], sidx.at[r], isems[r]).wait()

    def _didx_start(j, rd):
        pltpu.async_copy(dst_hbm.at[w].at[j], didx.at[rd], dsems[rd])

    def _didx_wait(j, rd):
        pltpu.make_async_copy(dst_hbm.at[w].at[j], didx.at[rd], dsems[rd]).wait()

    def _g_start(r, b):
        pltpu.async_copy(h_hbm.at[sidx.at[r]], gbufs[b], gsems[b])

    def _g_wait(r, b):
        pltpu.make_async_copy(h_hbm.at[sidx.at[r]], gbufs[b], gsems[b]).wait()

    for j in range(4):
        _sidx_start(j, j)
    for j in range(3):
        _didx_start(j, j)

    def _zero(k, _):
        i = k // (D // 16)
        j = k % (D // 16)
        buf0[i, pl.ds(j * 16, 16)] = jnp.zeros((16,), jnp.float32)
        return 0

    lax.fori_loop(0, CHUNK * (D // 16), _zero, 0)
    for off, ln in ZSPANS:
        pltpu.sync_copy(
            buf0.at[pl.ds(0, ln)], agg_sh.at[pl.ds(s * ZROWS + off, ln)]
        )
    plsc.subcore_barrier()

    for j in range(2):
        _sidx_wait(j, j)
        _g_start(j, j % 3)

    def _step(j, p, do_sidx, do_didx, do_g):
        b = p % 3
        r = p % 4
        _g_wait(r, b)
        if do_g:
            r2 = (p + 2) % 4
            _sidx_wait(j + 2, r2)
            _g_start(r2, (p + 2) % 3)
        _didx_wait(j, b)
        pltpu.sync_copy(gbufs[b], agg_sh.at[didx.at[b]], add=True)
        if do_sidx:
            _sidx_start(j + 4, r)
        if do_didx:
            _didx_start(j + 3, b)

    def _block(g, _):
        j0 = 12 * g
        for p in range(12):
            _step(j0 + p, p, True, True, True)
        return 0

    n_blocks = (CPW - 8) // 12
    lax.fori_loop(0, n_blocks, _block, 0)
    j0 = n_blocks * 12
    for p in range(8):
        j = j0 + p
        _step(j, p, j + 4 < CPW, j + 3 < CPW, j + 2 < CPW)
    plsc.subcore_barrier()

    for off, ln in ZSPANS:
        r0 = s * ZROWS + off
        pltpu.sync_copy(agg_sh.at[pl.ds(r0, ln)], buf0.at[pl.ds(0, ln)])
        pltpu.sync_copy(buf0.at[pl.ds(0, ln)], out_hbm.at[c].at[pl.ds(r0, ln)])


_sc_agg = pl.kernel(
    _sc_agg_body,
    out_type=jax.ShapeDtypeStruct((NC, N_PAD, D), jnp.float32),
    mesh=plsc.VectorSubcoreMesh(
        core_axis_name="c", subcore_axis_name="s", num_cores=NC, num_subcores=NS
    ),
    scratch_types=[
        pltpu.VMEM((3, CHUNK), jnp.int32),
        pltpu.VMEM((4, CHUNK), jnp.int32),
        pltpu.VMEM((CHUNK, D), jnp.float32),
        pltpu.VMEM((CHUNK, D), jnp.float32),
        pltpu.VMEM((CHUNK, D), jnp.float32),
        pltpu.VMEM_SHARED((N_PAD, D), jnp.float32),
        pltpu.SemaphoreType.DMA,
        pltpu.SemaphoreType.DMA,
        pltpu.SemaphoreType.DMA,
        pltpu.SemaphoreType.DMA,
        pltpu.SemaphoreType.DMA,
        pltpu.SemaphoreType.DMA,
        pltpu.SemaphoreType.DMA,
        pltpu.SemaphoreType.DMA,
        pltpu.SemaphoreType.DMA,
        pltpu.SemaphoreType.DMA,
    ],
)


def _mlp_body(eps_ref, h_ref, a0_ref, a1_ref, w1_ref, b1_ref, w2_ref, b2_ref, o_ref):
    rst = h_ref[...] * (1.0 + eps_ref[0, 0]) + a0_ref[0] + a1_ref[0]
    hid = jnp.maximum(
        jnp.dot(rst, w1_ref[...], preferred_element_type=jnp.float32) + b1_ref[...], 0.0
    )
    o_ref[...] = jnp.dot(hid, w2_ref[...], preferred_element_type=jnp.float32) + b2_ref[...]


def _mlp_sum_body(eps_ref, h_ref, a0_ref, a1_ref, w1_ref, b1_ref, w2_ref, b2_ref, o_ref):
    rst = h_ref[...] * (1.0 + eps_ref[0, 0]) + a0_ref[0] + a1_ref[0]
    hid = jnp.maximum(
        jnp.dot(rst, w1_ref[...], preferred_element_type=jnp.float32) + b1_ref[...], 0.0
    )
    out = jnp.dot(hid, w2_ref[...], preferred_element_type=jnp.float32) + b2_ref[...]

    @pl.when(pl.program_id(0) == 0)
    def _():
        o_ref[...] = jnp.zeros_like(o_ref)

    o_ref[...] += jnp.sum(out, axis=0, keepdims=True)


_MLP_BLOCK = 1000
_MLP_GRID = N_NODES // _MLP_BLOCK


def _mlp_call(body, out_shape, out_spec):
    return pl.pallas_call(
        body,
        grid=(_MLP_GRID,),
        in_specs=[
            pl.BlockSpec(memory_space=pltpu.SMEM),
            pl.BlockSpec((_MLP_BLOCK, D), lambda i: (i, 0)),
            pl.BlockSpec((1, _MLP_BLOCK, D), lambda i: (0, i, 0)),
            pl.BlockSpec((1, _MLP_BLOCK, D), lambda i: (1, i, 0)),
            pl.BlockSpec((D, D), lambda i: (0, 0)),
            pl.BlockSpec((1, D), lambda i: (0, 0)),
            pl.BlockSpec((D, D), lambda i: (0, 0)),
            pl.BlockSpec((1, D), lambda i: (0, 0)),
        ],
        out_specs=out_spec,
        out_shape=out_shape,
    )


_mlp = _mlp_call(
    _mlp_body,
    jax.ShapeDtypeStruct((N_NODES, D), jnp.float32),
    pl.BlockSpec((_MLP_BLOCK, D), lambda i: (i, 0)),
)
_mlp_sum = _mlp_call(
    _mlp_sum_body,
    jax.ShapeDtypeStruct((1, D), jnp.float32),
    pl.BlockSpec((1, D), lambda i: (0, 0)),
)


@jax.jit
def kernel(feats, edge_index, W1, b1, W2, b2, eps):
    src = edge_index[0].astype(jnp.int32)
    dst = edge_index[1].astype(jnp.int32)
    epc = NS * EPW
    rpc = N_EDGES // NC
    ppc = epc - rpc
    pad_src = jnp.arange(ppc, dtype=jnp.int32) % N_NODES
    pad_dst = jnp.arange(ppc, dtype=jnp.int32) % (N_PAD - N_NODES) + N_NODES
    src = jnp.concatenate(
        [src[:rpc], pad_src, src[rpc:], pad_src]
    ).reshape(NC * NS, CPW, CHUNK)
    dst = jnp.concatenate(
        [dst[:rpc], pad_dst, dst[rpc:], pad_dst]
    ).reshape(NC * NS, CPW, CHUNK)

    h = feats
    for i in range(NUM_LAYERS):
        agg = _sc_agg(src, dst, h)
        eps_i = eps[i].reshape(1, 1)
        args = (eps_i, h, agg, agg, W1[i], b1[i].reshape(1, D), W2[i], b2[i].reshape(1, D))
        if i < NUM_LAYERS - 1:
            h = _mlp(*args)
        else:
            return _mlp_sum(*args)


# speedup vs baseline: 4.5045x; 1.0401x over previous
import functools

import jax
import jax.numpy as jnp
from jax import lax
from jax.experimental import pallas as pl
from jax.experimental.pallas import tpu as pltpu
from jax.experimental.pallas import tpu_sc as plsc

N_NODES = 10000
N_EDGES = 320000
D = 128
NUM_LAYERS = 3

NC = 2
NS = 16
CHUNK = 128
CPW = 80
EPW = CPW * CHUNK
E_PAD = NC * NS * EPW
N_PAD = 10112
ZROWS = N_PAD // NS
ZSPANS = ((0, 128), (128, 128), (256, 128), (384, 128), (512, 120))


def _sc_agg_body(
    ei_hbm, h_hbm, out_hbm,
    didx, sidx, buf0, buf1, buf2, agg_sh,
    g0, g1, g2, i0, i1, i2, i3, d0, d1, d2,
):
    isems = (i0, i1, i2, i3)
    dsems = (d0, d1, d2)
    gbufs = (buf0, buf1, buf2)
    gsems = (g0, g1, g2)
    c = lax.axis_index("c")
    s = lax.axis_index("s")
    w = c * NS + s
    src_hbm = ei_hbm.at[0]
    dst_hbm = ei_hbm.at[1]

    def _sidx_start(j, r):
        pltpu.async_copy(src_hbm.at[w].at[j], sidx.at[r], isems[r])

    def _sidx_wait(j, r):
        pltpu.make_async_copy(src_hbm.at[w].at[j], sidx.at[r], isems[r]).wait()

    def _didx_start(j, rd):
        pltpu.async_copy(dst_hbm.at[w].at[j], didx.at[rd], dsems[rd])

    def _didx_wait(j, rd):
        pltpu.make_async_copy(dst_hbm.at[w].at[j], didx.at[rd], dsems[rd]).wait()

    def _g_start(r, b):
        pltpu.async_copy(h_hbm.at[sidx.at[r]], gbufs[b], gsems[b])

    def _g_wait(r, b):
        pltpu.make_async_copy(h_hbm.at[sidx.at[r]], gbufs[b], gsems[b]).wait()

    for j in range(4):
        _sidx_start(j, j)
    for j in range(3):
        _didx_start(j, j)

    def _zero(k, _):
        i = k // (D // 16)
        j = k % (D // 16)
        buf0[i, pl.ds(j * 16, 16)] = jnp.zeros((16,), jnp.float32)
        return 0

    lax.fori_loop(0, CHUNK * (D // 16), _zero, 0)
    for off, ln in ZSPANS:
        pltpu.sync_copy(
            buf0.at[pl.ds(0, ln)], agg_sh.at[pl.ds(s * ZROWS + off, ln)]
        )
    plsc.subcore_barrier()

    for j in range(2):
        _sidx_wait(j, j)
        _g_start(j, j % 3)

    def _step(j, p, do_sidx, do_didx, do_g):
        b = p % 3
        r = p % 4
        _g_wait(r, b)
        if do_g:
            r2 = (p + 2) % 4
            _sidx_wait(j + 2, r2)
            _g_start(r2, (p + 2) % 3)
        _didx_wait(j, b)
        pltpu.sync_copy(gbufs[b], agg_sh.at[didx.at[b]], add=True)
        if do_sidx:
            _sidx_start(j + 4, r)
        if do_didx:
            _didx_start(j + 3, b)

    def _block(g, _):
        j0 = 12 * g
        for p in range(12):
            _step(j0 + p, p, True, True, True)
        return 0

    n_blocks = (CPW - 8) // 12
    lax.fori_loop(0, n_blocks, _block, 0)
    j0 = n_blocks * 12
    for p in range(8):
        j = j0 + p
        _step(j, p, j + 4 < CPW, j + 3 < CPW, j + 2 < CPW)
    plsc.subcore_barrier()

    for off, ln in ZSPANS:
        r0 = s * ZROWS + off
        pltpu.sync_copy(agg_sh.at[pl.ds(r0, ln)], buf0.at[pl.ds(0, ln)])
        pltpu.sync_copy(buf0.at[pl.ds(0, ln)], out_hbm.at[c].at[pl.ds(r0, ln)])


_sc_agg = pl.kernel(
    _sc_agg_body,
    out_type=jax.ShapeDtypeStruct((NC, N_PAD, D), jnp.float32),
    mesh=plsc.VectorSubcoreMesh(
        core_axis_name="c", subcore_axis_name="s", num_cores=NC, num_subcores=NS
    ),
    scratch_types=[
        pltpu.VMEM((3, CHUNK), jnp.int32),
        pltpu.VMEM((4, CHUNK), jnp.int32),
        pltpu.VMEM((CHUNK, D), jnp.float32),
        pltpu.VMEM((CHUNK, D), jnp.float32),
        pltpu.VMEM((CHUNK, D), jnp.float32),
        pltpu.VMEM_SHARED((N_PAD, D), jnp.float32),
        pltpu.SemaphoreType.DMA,
        pltpu.SemaphoreType.DMA,
        pltpu.SemaphoreType.DMA,
        pltpu.SemaphoreType.DMA,
        pltpu.SemaphoreType.DMA,
        pltpu.SemaphoreType.DMA,
        pltpu.SemaphoreType.DMA,
        pltpu.SemaphoreType.DMA,
        pltpu.SemaphoreType.DMA,
        pltpu.SemaphoreType.DMA,
    ],
)


def _mlp_body(eps_ref, h_ref, a0_ref, a1_ref, w1_ref, b1_ref, w2_ref, b2_ref, o_ref):
    rst = h_ref[...] * (1.0 + eps_ref[0, 0]) + a0_ref[0] + a1_ref[0]
    hid = jnp.maximum(
        jnp.dot(rst, w1_ref[...], preferred_element_type=jnp.float32) + b1_ref[...], 0.0
    )
    o_ref[...] = jnp.dot(hid, w2_ref[...], preferred_element_type=jnp.float32) + b2_ref[...]


def _mlp_sum_body(eps_ref, h_ref, a0_ref, a1_ref, w1_ref, b1_ref, w2_ref, b2_ref, o_ref):
    rst = h_ref[...] * (1.0 + eps_ref[0, 0]) + a0_ref[0] + a1_ref[0]
    hid = jnp.maximum(
        jnp.dot(rst, w1_ref[...], preferred_element_type=jnp.float32) + b1_ref[...], 0.0
    )
    out = jnp.dot(hid, w2_ref[...], preferred_element_type=jnp.float32) + b2_ref[...]

    @pl.when(pl.program_id(0) == 0)
    def _():
        o_ref[...] = jnp.zeros_like(o_ref)

    o_ref[...] += jnp.sum(out, axis=0, keepdims=True)


_MLP_BLOCK = 2000
_MLP_GRID = N_NODES // _MLP_BLOCK


def _mlp_call(body, out_shape, out_spec):
    return pl.pallas_call(
        body,
        grid=(_MLP_GRID,),
        in_specs=[
            pl.BlockSpec(memory_space=pltpu.SMEM),
            pl.BlockSpec((_MLP_BLOCK, D), lambda i: (i, 0)),
            pl.BlockSpec((1, _MLP_BLOCK, D), lambda i: (0, i, 0)),
            pl.BlockSpec((1, _MLP_BLOCK, D), lambda i: (1, i, 0)),
            pl.BlockSpec((D, D), lambda i: (0, 0)),
            pl.BlockSpec((1, D), lambda i: (0, 0)),
            pl.BlockSpec((D, D), lambda i: (0, 0)),
            pl.BlockSpec((1, D), lambda i: (0, 0)),
        ],
        out_specs=out_spec,
        out_shape=out_shape,
    )


_mlp = _mlp_call(
    _mlp_body,
    jax.ShapeDtypeStruct((N_NODES, D), jnp.float32),
    pl.BlockSpec((_MLP_BLOCK, D), lambda i: (i, 0)),
)
_mlp_sum = _mlp_call(
    _mlp_sum_body,
    jax.ShapeDtypeStruct((1, D), jnp.float32),
    pl.BlockSpec((1, D), lambda i: (0, 0)),
)


@jax.jit
def kernel(feats, edge_index, W1, b1, W2, b2, eps):
    epc = NS * EPW
    rpc = N_EDGES // NC
    ppc = epc - rpc
    ei = edge_index.astype(jnp.int32)
    pad = jnp.stack(
        [
            jnp.arange(ppc, dtype=jnp.int32) % N_NODES,
            jnp.arange(ppc, dtype=jnp.int32) % (N_PAD - N_NODES) + N_NODES,
        ]
    )
    ei = jnp.concatenate([ei[:, :rpc], pad, ei[:, rpc:], pad], axis=1).reshape(
        2, NC * NS, CPW, CHUNK
    )

    h = feats
    for i in range(NUM_LAYERS):
        agg = _sc_agg(ei, h)
        eps_i = eps[i].reshape(1, 1)
        args = (eps_i, h, agg, agg, W1[i], b1[i].reshape(1, D), W2[i], b2[i].reshape(1, D))
        if i < NUM_LAYERS - 1:
            h = _mlp(*args)
        else:
            return _mlp_sum(*args)


# speedup vs baseline: 4.5265x; 1.0049x over previous
import functools

import jax
import jax.numpy as jnp
from jax import lax
from jax.experimental import pallas as pl
from jax.experimental.pallas import tpu as pltpu
from jax.experimental.pallas import tpu_sc as plsc

N_NODES = 10000
N_EDGES = 320000
D = 128
NUM_LAYERS = 3

NC = 2
NS = 16
CHUNK = 128
CPW = 80
EPW = CPW * CHUNK
E_PAD = NC * NS * EPW
N_PAD = 10112
ZROWS = N_PAD // NS
ZSPANS = ((0, 128), (128, 128), (256, 128), (384, 128), (512, 120))


def _sc_agg_body(
    ei_hbm, h_hbm, out_hbm,
    didx, sidx, buf0, buf1, buf2, agg_sh,
    g0, g1, g2, i0, i1, i2, i3, d0, d1, d2,
):
    isems = (i0, i1, i2, i3)
    dsems = (d0, d1, d2)
    gbufs = (buf0, buf1, buf2)
    gsems = (g0, g1, g2)
    c = lax.axis_index("c")
    s = lax.axis_index("s")
    w = c * NS + s
    src_hbm = ei_hbm.at[0]
    dst_hbm = ei_hbm.at[1]

    def _sidx_start(j, r):
        pltpu.async_copy(src_hbm.at[w].at[j], sidx.at[r], isems[r])

    def _sidx_wait(j, r):
        pltpu.make_async_copy(src_hbm.at[w].at[j], sidx.at[r], isems[r]).wait()

    def _didx_start(j, rd):
        pltpu.async_copy(dst_hbm.at[w].at[j], didx.at[rd], dsems[rd])

    def _didx_wait(j, rd):
        pltpu.make_async_copy(dst_hbm.at[w].at[j], didx.at[rd], dsems[rd]).wait()

    def _g_start(r, b):
        pltpu.async_copy(h_hbm.at[sidx.at[r]], gbufs[b], gsems[b])

    def _g_wait(r, b):
        pltpu.make_async_copy(h_hbm.at[sidx.at[r]], gbufs[b], gsems[b]).wait()

    for j in range(4):
        _sidx_start(j, j)
    for j in range(3):
        _didx_start(j, j)

    def _zero(k, _):
        i = k // (D // 16)
        j = k % (D // 16)
        buf0[i, pl.ds(j * 16, 16)] = jnp.zeros((16,), jnp.float32)
        return 0

    lax.fori_loop(0, CHUNK * (D // 16), _zero, 0)
    zsems = (g0, g1, g2)
    for k, (off, ln) in enumerate(ZSPANS):
        if k == 3:
            for kk in range(3):
                o2, l2 = ZSPANS[kk]
                pltpu.make_async_copy(
                    buf0.at[pl.ds(0, l2)],
                    agg_sh.at[pl.ds(s * ZROWS + o2, l2)],
                    zsems[kk],
                ).wait()
        pltpu.async_copy(
            buf0.at[pl.ds(0, ln)],
            agg_sh.at[pl.ds(s * ZROWS + off, ln)],
            zsems[k % 3],
        )
    for kk in range(3, 5):
        o2, l2 = ZSPANS[kk]
        pltpu.make_async_copy(
            buf0.at[pl.ds(0, l2)],
            agg_sh.at[pl.ds(s * ZROWS + o2, l2)],
            zsems[kk % 3],
        ).wait()
    plsc.subcore_barrier()

    for j in range(2):
        _sidx_wait(j, j)
        _g_start(j, j % 3)

    def _step(j, p, do_sidx, do_didx, do_g):
        b = p % 3
        r = p % 4
        _g_wait(r, b)
        if do_g:
            r2 = (p + 2) % 4
            _sidx_wait(j + 2, r2)
            _g_start(r2, (p + 2) % 3)
        _didx_wait(j, b)
        pltpu.sync_copy(gbufs[b], agg_sh.at[didx.at[b]], add=True)
        if do_sidx:
            _sidx_start(j + 4, r)
        if do_didx:
            _didx_start(j + 3, b)

    def _block(g, _):
        j0 = 12 * g
        for p in range(12):
            _step(j0 + p, p, True, True, True)
        return 0

    n_blocks = (CPW - 8) // 12
    lax.fori_loop(0, n_blocks, _block, 0)
    j0 = n_blocks * 12
    for p in range(8):
        j = j0 + p
        _step(j, p, j + 4 < CPW, j + 3 < CPW, j + 2 < CPW)
    plsc.subcore_barrier()

    pltpu.async_copy(
        agg_sh.at[pl.ds(s * ZROWS, ZROWS)],
        out_hbm.at[c].at[pl.ds(s * ZROWS, ZROWS)],
        g0,
    )
    pltpu.make_async_copy(
        agg_sh.at[pl.ds(s * ZROWS, ZROWS)],
        out_hbm.at[c].at[pl.ds(s * ZROWS, ZROWS)],
        g0,
    ).wait()


_sc_agg = pl.kernel(
    _sc_agg_body,
    out_type=jax.ShapeDtypeStruct((NC, N_PAD, D), jnp.float32),
    mesh=plsc.VectorSubcoreMesh(
        core_axis_name="c", subcore_axis_name="s", num_cores=NC, num_subcores=NS
    ),
    scratch_types=[
        pltpu.VMEM((3, CHUNK), jnp.int32),
        pltpu.VMEM((4, CHUNK), jnp.int32),
        pltpu.VMEM((CHUNK, D), jnp.float32),
        pltpu.VMEM((CHUNK, D), jnp.float32),
        pltpu.VMEM((CHUNK, D), jnp.float32),
        pltpu.VMEM_SHARED((N_PAD, D), jnp.float32),
        pltpu.SemaphoreType.DMA,
        pltpu.SemaphoreType.DMA,
        pltpu.SemaphoreType.DMA,
        pltpu.SemaphoreType.DMA,
        pltpu.SemaphoreType.DMA,
        pltpu.SemaphoreType.DMA,
        pltpu.SemaphoreType.DMA,
        pltpu.SemaphoreType.DMA,
        pltpu.SemaphoreType.DMA,
        pltpu.SemaphoreType.DMA,
    ],
)


def _mlp_body(eps_ref, h_ref, a0_ref, a1_ref, w1_ref, b1_ref, w2_ref, b2_ref, o_ref):
    rst = h_ref[...] * (1.0 + eps_ref[0, 0]) + a0_ref[0] + a1_ref[0]
    hid = jnp.maximum(
        jnp.dot(rst, w1_ref[...], preferred_element_type=jnp.float32) + b1_ref[...], 0.0
    )
    o_ref[...] = jnp.dot(hid, w2_ref[...], preferred_element_type=jnp.float32) + b2_ref[...]


def _mlp_sum_body(eps_ref, h_ref, a0_ref, a1_ref, w1_ref, b1_ref, w2_ref, b2_ref, o_ref):
    rst = h_ref[...] * (1.0 + eps_ref[0, 0]) + a0_ref[0] + a1_ref[0]
    hid = jnp.maximum(
        jnp.dot(rst, w1_ref[...], preferred_element_type=jnp.float32) + b1_ref[...], 0.0
    )
    out = jnp.dot(hid, w2_ref[...], preferred_element_type=jnp.float32) + b2_ref[...]

    @pl.when(pl.program_id(0) == 0)
    def _():
        o_ref[...] = jnp.zeros_like(o_ref)

    o_ref[...] += jnp.sum(out, axis=0, keepdims=True)


_MLP_BLOCK = 2000
_MLP_GRID = N_NODES // _MLP_BLOCK


def _mlp_call(body, out_shape, out_spec):
    return pl.pallas_call(
        body,
        grid=(_MLP_GRID,),
        in_specs=[
            pl.BlockSpec(memory_space=pltpu.SMEM),
            pl.BlockSpec((_MLP_BLOCK, D), lambda i: (i, 0)),
            pl.BlockSpec((1, _MLP_BLOCK, D), lambda i: (0, i, 0)),
            pl.BlockSpec((1, _MLP_BLOCK, D), lambda i: (1, i, 0)),
            pl.BlockSpec((D, D), lambda i: (0, 0)),
            pl.BlockSpec((1, D), lambda i: (0, 0)),
            pl.BlockSpec((D, D), lambda i: (0, 0)),
            pl.BlockSpec((1, D), lambda i: (0, 0)),
        ],
        out_specs=out_spec,
        out_shape=out_shape,
    )


_mlp = _mlp_call(
    _mlp_body,
    jax.ShapeDtypeStruct((N_NODES, D), jnp.float32),
    pl.BlockSpec((_MLP_BLOCK, D), lambda i: (i, 0)),
)
_mlp_sum = _mlp_call(
    _mlp_sum_body,
    jax.ShapeDtypeStruct((1, D), jnp.float32),
    pl.BlockSpec((1, D), lambda i: (0, 0)),
)


@jax.jit
def kernel(feats, edge_index, W1, b1, W2, b2, eps):
    epc = NS * EPW
    rpc = N_EDGES // NC
    ppc = epc - rpc
    ei = edge_index.astype(jnp.int32)
    pad = jnp.stack(
        [
            jnp.arange(ppc, dtype=jnp.int32) % N_NODES,
            jnp.arange(ppc, dtype=jnp.int32) % (N_PAD - N_NODES) + N_NODES,
        ]
    )
    ei = jnp.concatenate([ei[:, :rpc], pad, ei[:, rpc:], pad], axis=1).reshape(
        2, NC * NS, CPW, CHUNK
    )

    h = feats
    for i in range(NUM_LAYERS):
        agg = _sc_agg(ei, h)
        eps_i = eps[i].reshape(1, 1)
        args = (eps_i, h, agg, agg, W1[i], b1[i].reshape(1, D), W2[i], b2[i].reshape(1, D))
        if i < NUM_LAYERS - 1:
            h = _mlp(*args)
        else:
            return _mlp_sum(*args)


# speedup vs baseline: 4.6944x; 1.0371x over previous
import functools

import jax
import jax.numpy as jnp
from jax import lax
from jax.experimental import pallas as pl
from jax.experimental.pallas import tpu as pltpu
from jax.experimental.pallas import tpu_sc as plsc

N_NODES = 10000
N_EDGES = 320000
D = 128
NUM_LAYERS = 3

NC = 2
NS = 16
CHUNK = 128
NCHUNKS = N_EDGES // CHUNK
CPW = NCHUNKS // (NC * NS)
XTRA = NCHUNKS - CPW * NC * NS
N_PAD = 10112
ZROWS = N_PAD // NS
ZSPANS = ((0, 128), (128, 128), (256, 128), (384, 128), (512, 120))


def _sc_agg_body(
    ei_hbm, h_hbm, out_hbm,
    didx, sidx, buf0, buf1, buf2, agg_sh,
    g0, g1, g2, i0, i1, i2, i3, d0, d1, d2,
):
    isems = (i0, i1, i2, i3)
    dsems = (d0, d1, d2)
    gbufs = (buf0, buf1, buf2)
    gsems = (g0, g1, g2)
    c = lax.axis_index("c")
    s = lax.axis_index("s")
    w = c * NS + s
    base = CPW * w + jnp.minimum(w, XTRA)
    src_hbm = ei_hbm.at[0]
    dst_hbm = ei_hbm.at[1]

    def _sidx_start(j, r):
        pltpu.async_copy(src_hbm.at[base + j], sidx.at[r], isems[r])

    def _sidx_wait(j, r):
        pltpu.make_async_copy(src_hbm.at[base + j], sidx.at[r], isems[r]).wait()

    def _didx_start(j, rd):
        pltpu.async_copy(dst_hbm.at[base + j], didx.at[rd], dsems[rd])

    def _didx_wait(j, rd):
        pltpu.make_async_copy(dst_hbm.at[base + j], didx.at[rd], dsems[rd]).wait()

    def _g_start(r, b):
        pltpu.async_copy(h_hbm.at[sidx.at[r]], gbufs[b], gsems[b])

    def _g_wait(r, b):
        pltpu.make_async_copy(h_hbm.at[sidx.at[r]], gbufs[b], gsems[b]).wait()

    for j in range(4):
        _sidx_start(j, j)
    for j in range(3):
        _didx_start(j, j)

    def _zero(k, _):
        i = k // (D // 16)
        j = k % (D // 16)
        buf0[i, pl.ds(j * 16, 16)] = jnp.zeros((16,), jnp.float32)
        return 0

    lax.fori_loop(0, CHUNK * (D // 16), _zero, 0)
    zsems = (g0, g1, g2)
    for k, (off, ln) in enumerate(ZSPANS):
        if k == 3:
            for kk in range(3):
                o2, l2 = ZSPANS[kk]
                pltpu.make_async_copy(
                    buf0.at[pl.ds(0, l2)],
                    agg_sh.at[pl.ds(s * ZROWS + o2, l2)],
                    zsems[kk],
                ).wait()
        pltpu.async_copy(
            buf0.at[pl.ds(0, ln)],
            agg_sh.at[pl.ds(s * ZROWS + off, ln)],
            zsems[k % 3],
        )
    for kk in range(3, 5):
        o2, l2 = ZSPANS[kk]
        pltpu.make_async_copy(
            buf0.at[pl.ds(0, l2)],
            agg_sh.at[pl.ds(s * ZROWS + o2, l2)],
            zsems[kk % 3],
        ).wait()
    plsc.subcore_barrier()

    for j in range(2):
        _sidx_wait(j, j)
        _g_start(j, j % 3)

    def _step(j, p, do_sidx, do_didx, do_g):
        b = p % 3
        r = p % 4
        _g_wait(r, b)
        if do_g:
            r2 = (p + 2) % 4
            _sidx_wait(j + 2, r2)
            _g_start(r2, (p + 2) % 3)
        _didx_wait(j, b)
        pltpu.sync_copy(gbufs[b], agg_sh.at[didx.at[b]], add=True)
        if do_sidx:
            _sidx_start(j + 4, r)
        if do_didx:
            _didx_start(j + 3, b)

    def _block(g, _):
        j0 = 12 * g
        for p in range(12):
            _step(j0 + p, p, True, True, True)
        return 0

    n_blocks = (CPW - 6) // 12
    lax.fori_loop(0, n_blocks, _block, 0)
    j0 = n_blocks * 12
    for p in range(CPW - j0):
        j = j0 + p
        _step(j, p, j + 4 < CPW, j + 3 < CPW, j + 2 < CPW)

    @pl.when(w < XTRA)
    def _extra():
        _sidx_start(CPW, 0)
        _didx_start(CPW, 0)
        _sidx_wait(CPW, 0)
        _g_start(0, 0)
        _g_wait(0, 0)
        _didx_wait(CPW, 0)
        pltpu.sync_copy(gbufs[0], agg_sh.at[didx.at[0]], add=True)

    plsc.subcore_barrier()

    pltpu.async_copy(
        agg_sh.at[pl.ds(s * ZROWS, ZROWS)],
        out_hbm.at[c].at[pl.ds(s * ZROWS, ZROWS)],
        g0,
    )
    pltpu.make_async_copy(
        agg_sh.at[pl.ds(s * ZROWS, ZROWS)],
        out_hbm.at[c].at[pl.ds(s * ZROWS, ZROWS)],
        g0,
    ).wait()


_sc_agg = pl.kernel(
    _sc_agg_body,
    out_type=jax.ShapeDtypeStruct((NC, N_PAD, D), jnp.float32),
    mesh=plsc.VectorSubcoreMesh(
        core_axis_name="c", subcore_axis_name="s", num_cores=NC, num_subcores=NS
    ),
    scratch_types=[
        pltpu.VMEM((3, CHUNK), jnp.int32),
        pltpu.VMEM((4, CHUNK), jnp.int32),
        pltpu.VMEM((CHUNK, D), jnp.float32),
        pltpu.VMEM((CHUNK, D), jnp.float32),
        pltpu.VMEM((CHUNK, D), jnp.float32),
        pltpu.VMEM_SHARED((N_PAD, D), jnp.float32),
        pltpu.SemaphoreType.DMA,
        pltpu.SemaphoreType.DMA,
        pltpu.SemaphoreType.DMA,
        pltpu.SemaphoreType.DMA,
        pltpu.SemaphoreType.DMA,
        pltpu.SemaphoreType.DMA,
        pltpu.SemaphoreType.DMA,
        pltpu.SemaphoreType.DMA,
        pltpu.SemaphoreType.DMA,
        pltpu.SemaphoreType.DMA,
    ],
)


def _mlp_body(eps_ref, h_ref, a0_ref, a1_ref, w1_ref, b1_ref, w2_ref, b2_ref, o_ref):
    rst = h_ref[...] * (1.0 + eps_ref[0, 0]) + a0_ref[0] + a1_ref[0]
    hid = jnp.maximum(
        jnp.dot(rst, w1_ref[...], preferred_element_type=jnp.float32) + b1_ref[...], 0.0
    )
    o_ref[...] = jnp.dot(hid, w2_ref[...], preferred_element_type=jnp.float32) + b2_ref[...]


def _mlp_sum_body(eps_ref, h_ref, a0_ref, a1_ref, w1_ref, b1_ref, w2_ref, b2_ref, o_ref):
    rst = h_ref[...] * (1.0 + eps_ref[0, 0]) + a0_ref[0] + a1_ref[0]
    hid = jnp.maximum(
        jnp.dot(rst, w1_ref[...], preferred_element_type=jnp.float32) + b1_ref[...], 0.0
    )
    out = jnp.dot(hid, w2_ref[...], preferred_element_type=jnp.float32) + b2_ref[...]

    @pl.when(pl.program_id(0) == 0)
    def _():
        o_ref[...] = jnp.zeros_like(o_ref)

    o_ref[...] += jnp.sum(out, axis=0, keepdims=True)


_MLP_BLOCK = 2000
_MLP_GRID = N_NODES // _MLP_BLOCK


def _mlp_call(body, out_shape, out_spec):
    return pl.pallas_call(
        body,
        grid=(_MLP_GRID,),
        in_specs=[
            pl.BlockSpec(memory_space=pltpu.SMEM),
            pl.BlockSpec((_MLP_BLOCK, D), lambda i: (i, 0)),
            pl.BlockSpec((1, _MLP_BLOCK, D), lambda i: (0, i, 0)),
            pl.BlockSpec((1, _MLP_BLOCK, D), lambda i: (1, i, 0)),
            pl.BlockSpec((D, D), lambda i: (0, 0)),
            pl.BlockSpec((1, D), lambda i: (0, 0)),
            pl.BlockSpec((D, D), lambda i: (0, 0)),
            pl.BlockSpec((1, D), lambda i: (0, 0)),
        ],
        out_specs=out_spec,
        out_shape=out_shape,
    )


_mlp = _mlp_call(
    _mlp_body,
    jax.ShapeDtypeStruct((N_NODES, D), jnp.float32),
    pl.BlockSpec((_MLP_BLOCK, D), lambda i: (i, 0)),
)
_mlp_sum = _mlp_call(
    _mlp_sum_body,
    jax.ShapeDtypeStruct((1, D), jnp.float32),
    pl.BlockSpec((1, D), lambda i: (0, 0)),
)


@jax.jit
def kernel(feats, edge_index, W1, b1, W2, b2, eps):
    ei = edge_index.astype(jnp.int32).reshape(2, NCHUNKS, CHUNK)

    h = feats
    for i in range(NUM_LAYERS):
        agg = _sc_agg(ei, h)
        eps_i = eps[i].reshape(1, 1)
        args = (eps_i, h, agg, agg, W1[i], b1[i].reshape(1, D), W2[i], b2[i].reshape(1, D))
        if i < NUM_LAYERS - 1:
            h = _mlp(*args)
        else:
            return _mlp_sum(*args)


# speedup vs baseline: 4.7054x; 1.0023x over previous
import functools

import jax
import jax.numpy as jnp
from jax import lax
from jax.experimental import pallas as pl
from jax.experimental.pallas import tpu as pltpu
from jax.experimental.pallas import tpu_sc as plsc

N_NODES = 10000
N_EDGES = 320000
D = 128
NUM_LAYERS = 3

NC = 2
NS = 16
CHUNK = 128
NCHUNKS = N_EDGES // CHUNK
CPW = NCHUNKS // (NC * NS)
XTRA = NCHUNKS - CPW * NC * NS
N_PAD = 10112
ZROWS = N_PAD // NS
ZSPANS = ((0, 128), (128, 128), (256, 128), (384, 128), (512, 120))


def _sc_agg_body(
    ei_hbm, h_hbm, out_hbm,
    didx, sidx, buf0, buf1, buf2, agg_sh,
    g0, g1, g2, i0, i1, i2, i3, d0, d1, d2,
):
    isems = (i0, i1, i2, i3)
    dsems = (d0, d1, d2)
    gbufs = (buf0, buf1, buf2)
    gsems = (g0, g1, g2)
    c = lax.axis_index("c")
    s = lax.axis_index("s")
    w = c * NS + s
    is_xtra = w % 8 == 0
    base = CPW * w + (w + 7) // 8
    src_hbm = ei_hbm.at[0]
    dst_hbm = ei_hbm.at[1]

    def _sidx_start(j, r):
        pltpu.async_copy(src_hbm.at[pl.ds((base + j) * CHUNK, CHUNK)], sidx.at[r], isems[r])

    def _sidx_wait(j, r):
        pltpu.make_async_copy(
            src_hbm.at[pl.ds((base + j) * CHUNK, CHUNK)], sidx.at[r], isems[r]
        ).wait()

    def _didx_start(j, rd):
        pltpu.async_copy(dst_hbm.at[pl.ds((base + j) * CHUNK, CHUNK)], didx.at[rd], dsems[rd])

    def _didx_wait(j, rd):
        pltpu.make_async_copy(
            dst_hbm.at[pl.ds((base + j) * CHUNK, CHUNK)], didx.at[rd], dsems[rd]
        ).wait()

    def _g_start(r, b):
        pltpu.async_copy(h_hbm.at[sidx.at[r]], gbufs[b], gsems[b])

    def _g_wait(r, b):
        pltpu.make_async_copy(h_hbm.at[sidx.at[r]], gbufs[b], gsems[b]).wait()

    for j in range(4):
        _sidx_start(j, j)
    for j in range(3):
        _didx_start(j, j)

    def _zero(k, _):
        i = k // (D // 16)
        j = k % (D // 16)
        buf0[i, pl.ds(j * 16, 16)] = jnp.zeros((16,), jnp.float32)
        return 0

    lax.fori_loop(0, CHUNK * (D // 16), _zero, 0)
    zsems = (g0, g1, g2)
    for k, (off, ln) in enumerate(ZSPANS):
        if k == 3:
            for kk in range(3):
                o2, l2 = ZSPANS[kk]
                pltpu.make_async_copy(
                    buf0.at[pl.ds(0, l2)],
                    agg_sh.at[pl.ds(s * ZROWS + o2, l2)],
                    zsems[kk],
                ).wait()
        pltpu.async_copy(
            buf0.at[pl.ds(0, ln)],
            agg_sh.at[pl.ds(s * ZROWS + off, ln)],
            zsems[k % 3],
        )
    for kk in range(3, 5):
        o2, l2 = ZSPANS[kk]
        pltpu.make_async_copy(
            buf0.at[pl.ds(0, l2)],
            agg_sh.at[pl.ds(s * ZROWS + o2, l2)],
            zsems[kk % 3],
        ).wait()
    plsc.subcore_barrier()

    for j in range(2):
        _sidx_wait(j, j)
        _g_start(j, j % 3)

    def _step(j, p, do_sidx, do_didx, do_g):
        b = p % 3
        r = p % 4
        _g_wait(r, b)
        if do_g:
            r2 = (p + 2) % 4
            _sidx_wait(j + 2, r2)
            _g_start(r2, (p + 2) % 3)
        _didx_wait(j, b)
        pltpu.sync_copy(gbufs[b], agg_sh.at[didx.at[b]], add=True)
        if do_sidx:
            _sidx_start(j + 4, r)
        if do_didx:
            _didx_start(j + 3, b)

    def _block(g, _):
        j0 = 12 * g
        for p in range(12):
            _step(j0 + p, p, True, True, True)
        return 0

    n_blocks = (CPW - 6) // 12
    lax.fori_loop(0, n_blocks, _block, 0)
    j0 = n_blocks * 12
    for p in range(CPW - j0):
        j = j0 + p
        _step(j, p, j + 4 < CPW, j + 3 < CPW, j + 2 < CPW)

    @pl.when(is_xtra)
    def _extra():
        _sidx_start(CPW, 0)
        _didx_start(CPW, 0)
        _sidx_wait(CPW, 0)
        _g_start(0, 0)
        _g_wait(0, 0)
        _didx_wait(CPW, 0)
        pltpu.sync_copy(gbufs[0], agg_sh.at[didx.at[0]], add=True)

    plsc.subcore_barrier()

    pltpu.async_copy(
        agg_sh.at[pl.ds(s * ZROWS, ZROWS)],
        out_hbm.at[c].at[pl.ds(s * ZROWS, ZROWS)],
        g0,
    )
    pltpu.make_async_copy(
        agg_sh.at[pl.ds(s * ZROWS, ZROWS)],
        out_hbm.at[c].at[pl.ds(s * ZROWS, ZROWS)],
        g0,
    ).wait()


_sc_agg = pl.kernel(
    _sc_agg_body,
    out_type=jax.ShapeDtypeStruct((NC, N_PAD, D), jnp.float32),
    mesh=plsc.VectorSubcoreMesh(
        core_axis_name="c", subcore_axis_name="s", num_cores=NC, num_subcores=NS
    ),
    scratch_types=[
        pltpu.VMEM((3, CHUNK), jnp.int32),
        pltpu.VMEM((4, CHUNK), jnp.int32),
        pltpu.VMEM((CHUNK, D), jnp.float32),
        pltpu.VMEM((CHUNK, D), jnp.float32),
        pltpu.VMEM((CHUNK, D), jnp.float32),
        pltpu.VMEM_SHARED((N_PAD, D), jnp.float32),
        pltpu.SemaphoreType.DMA,
        pltpu.SemaphoreType.DMA,
        pltpu.SemaphoreType.DMA,
        pltpu.SemaphoreType.DMA,
        pltpu.SemaphoreType.DMA,
        pltpu.SemaphoreType.DMA,
        pltpu.SemaphoreType.DMA,
        pltpu.SemaphoreType.DMA,
        pltpu.SemaphoreType.DMA,
        pltpu.SemaphoreType.DMA,
    ],
)


def _mlp_body(eps_ref, h_ref, a0_ref, a1_ref, w1_ref, b1_ref, w2_ref, b2_ref, o_ref):
    rst = h_ref[...] * (1.0 + eps_ref[0, 0]) + a0_ref[0] + a1_ref[0]
    hid = jnp.maximum(
        jnp.dot(rst, w1_ref[...], preferred_element_type=jnp.float32) + b1_ref[...], 0.0
    )
    o_ref[...] = jnp.dot(hid, w2_ref[...], preferred_element_type=jnp.float32) + b2_ref[...]


def _mlp_sum_body(eps_ref, h_ref, a0_ref, a1_ref, w1_ref, b1_ref, w2_ref, b2_ref, o_ref):
    rst = h_ref[...] * (1.0 + eps_ref[0, 0]) + a0_ref[0] + a1_ref[0]
    hid = jnp.maximum(
        jnp.dot(rst, w1_ref[...], preferred_element_type=jnp.float32) + b1_ref[...], 0.0
    )
    out = jnp.dot(hid, w2_ref[...], preferred_element_type=jnp.float32) + b2_ref[...]

    @pl.when(pl.program_id(0) == 0)
    def _():
        o_ref[...] = jnp.zeros_like(o_ref)

    o_ref[...] += jnp.sum(out, axis=0, keepdims=True)


_MLP_BLOCK = 2000
_MLP_GRID = N_NODES // _MLP_BLOCK


def _mlp_call(body, out_shape, out_spec):
    return pl.pallas_call(
        body,
        grid=(_MLP_GRID,),
        in_specs=[
            pl.BlockSpec(memory_space=pltpu.SMEM),
            pl.BlockSpec((_MLP_BLOCK, D), lambda i: (i, 0)),
            pl.BlockSpec((1, _MLP_BLOCK, D), lambda i: (0, i, 0)),
            pl.BlockSpec((1, _MLP_BLOCK, D), lambda i: (1, i, 0)),
            pl.BlockSpec((D, D), lambda i: (0, 0)),
            pl.BlockSpec((1, D), lambda i: (0, 0)),
            pl.BlockSpec((D, D), lambda i: (0, 0)),
            pl.BlockSpec((1, D), lambda i: (0, 0)),
        ],
        out_specs=out_spec,
        out_shape=out_shape,
    )


_mlp = _mlp_call(
    _mlp_body,
    jax.ShapeDtypeStruct((N_NODES, D), jnp.float32),
    pl.BlockSpec((_MLP_BLOCK, D), lambda i: (i, 0)),
)
_mlp_sum = _mlp_call(
    _mlp_sum_body,
    jax.ShapeDtypeStruct((1, D), jnp.float32),
    pl.BlockSpec((1, D), lambda i: (0, 0)),
)


@jax.jit
def kernel(feats, edge_index, W1, b1, W2, b2, eps):
    ei = edge_index.astype(jnp.int32)

    h = feats
    for i in range(NUM_LAYERS):
        agg = _sc_agg(ei, h)
        eps_i = eps[i].reshape(1, 1)
        args = (eps_i, h, agg, agg, W1[i], b1[i].reshape(1, D), W2[i], b2[i].reshape(1, D))
        if i < NUM_LAYERS - 1:
            h = _mlp(*args)
        else:
            return _mlp_sum(*args)
